# initial kernel scaffold (unmeasured)
import functools

import jax
import jax.numpy as jnp
from jax import lax
from jax.experimental import pallas as pl
from jax.experimental.pallas import tpu as pltpu

N_DEV = 32
M = 4096
N_OUT = 8192
CHUNK = M // N_DEV


def _body(x_ref, w_ref, out_ref, comm, send_sems, recv_sems, credit_sem,
          store_sem, stage):
    me = lax.axis_index("i")
    right = lax.rem(me + 1, N_DEV)
    left = lax.rem(me + N_DEV - 1, N_DEV)

    bsem = pltpu.get_barrier_semaphore()
    for nbr in (left, right):
        pl.semaphore_signal(bsem, inc=1, device_id=(nbr,),
                            device_id_type=pl.DeviceIdType.MESH)
    pl.semaphore_wait(bsem, 2)

    def local_partial(c):
        return jnp.dot(x_ref[pl.ds(c * CHUNK, CHUNK), :], w_ref[...],
                       preferred_element_type=jnp.float32)

    def store_chunk(c, slot):
        stage[...] = jnp.maximum(comm[slot].astype(jnp.float32), 0.0)
        cp = pltpu.make_async_copy(
            stage, out_ref.at[pl.ds(c * CHUNK, CHUNK), :], store_sem)
        cp.start()
        cp.wait()

    def hop(k, process):
        send_slot = k % 2
        recv_slot = (k + 1) % 2
        rdma = pltpu.make_async_remote_copy(
            src_ref=comm.at[send_slot],
            dst_ref=comm.at[recv_slot],
            send_sem=send_sems.at[send_slot],
            recv_sem=recv_sems.at[recv_slot],
            device_id=(right,),
            device_id_type=pl.DeviceIdType.MESH,
        )
        if k > 0:
            pl.semaphore_wait(credit_sem, 1)
        rdma.start()
        rdma.wait()
        process(recv_slot)
        if k < 2 * (N_DEV - 1) - 1:
            pl.semaphore_signal(credit_sem, inc=1, device_id=(left,),
                                device_id_type=pl.DeviceIdType.MESH)

    comm[0] = local_partial(me).astype(jnp.bfloat16)
    for s in range(N_DEV - 1):
        c_recv = lax.rem(me - (s + 1) + 2 * N_DEV, N_DEV)

        def rs_process(recv_slot, c=c_recv):
            acc = comm[recv_slot].astype(jnp.float32) + local_partial(c)
            comm[recv_slot] = acc.astype(jnp.bfloat16)

        hop(s, rs_process)

    own = lax.rem(me + 1, N_DEV)
    store_chunk(own, (N_DEV - 1) % 2)

    for t in range(N_DEV - 1):
        k = (N_DEV - 1) + t
        c_recv = lax.rem(me - t + 2 * N_DEV, N_DEV)

        def ag_process(recv_slot, c=c_recv):
            store_chunk(c, recv_slot)

        hop(k, ag_process)


def kernel(x, w_mat):
    out_shape = jax.ShapeDtypeStruct((M, N_OUT), jnp.float32)
    return pl.pallas_call(
        _body,
        out_shape=out_shape,
        in_specs=[
            pl.BlockSpec(memory_space=pltpu.MemorySpace.VMEM),
            pl.BlockSpec(memory_space=pltpu.MemorySpace.VMEM),
        ],
        out_specs=pl.BlockSpec(memory_space=pltpu.MemorySpace.ANY),
        scratch_shapes=[
            pltpu.VMEM((2, CHUNK, N_OUT), jnp.bfloat16),
            pltpu.SemaphoreType.DMA((2,)),
            pltpu.SemaphoreType.DMA((2,)),
            pltpu.SemaphoreType.REGULAR,
            pltpu.SemaphoreType.DMA,
            pltpu.VMEM((CHUNK, N_OUT), jnp.float32),
        ],
        compiler_params=pltpu.CompilerParams(collective_id=0),
    )(x, w_mat)


# baseline (device time: 2017682 ns/iter reference)
import functools

import jax
import jax.numpy as jnp
from jax import lax
from jax.experimental import pallas as pl
from jax.experimental.pallas import tpu as pltpu

N_DEV = 32
M = 4096
N_OUT = 8192
CHUNK = M // N_DEV


def _body(x_ref, w_ref, out_ref, comm, send_sems, recv_sems, credit_sem,
          store_sem, stage):
    me = lax.axis_index("i")
    right = lax.rem(me + 1, N_DEV)
    left = lax.rem(me + N_DEV - 1, N_DEV)

    bsem = pltpu.get_barrier_semaphore()
    for nbr in (left, right):
        pl.semaphore_signal(bsem, inc=1, device_id=(nbr,),
                            device_id_type=pl.DeviceIdType.MESH)
    pl.semaphore_wait(bsem, 2)

    def local_partial(c):
        return jnp.dot(x_ref[pl.ds(c * CHUNK, CHUNK), :], w_ref[...],
                       preferred_element_type=jnp.float32)

    def store_chunk(c, slot):
        stage[...] = jnp.maximum(comm[slot].astype(jnp.float32), 0.0)
        cp = pltpu.make_async_copy(
            stage, out_ref.at[pl.ds(c * CHUNK, CHUNK), :], store_sem)
        cp.start()
        cp.wait()

    def hop(k, process):
        send_slot = k % 2
        recv_slot = (k + 1) % 2
        rdma = pltpu.make_async_remote_copy(
            src_ref=comm.at[send_slot],
            dst_ref=comm.at[recv_slot],
            send_sem=send_sems.at[send_slot],
            recv_sem=recv_sems.at[recv_slot],
            device_id=(right,),
            device_id_type=pl.DeviceIdType.MESH,
        )
        if k > 0:
            pl.semaphore_wait(credit_sem, 1)
        rdma.start()
        rdma.wait()
        process(recv_slot)
        if k < 2 * (N_DEV - 1) - 1:
            pl.semaphore_signal(credit_sem, inc=1, device_id=(left,),
                                device_id_type=pl.DeviceIdType.MESH)

    comm[0] = local_partial(me).astype(jnp.bfloat16)
    for s in range(N_DEV - 1):
        c_recv = lax.rem(me - (s + 1) + 2 * N_DEV, N_DEV)

        def rs_process(recv_slot, c=c_recv):
            acc = comm[recv_slot].astype(jnp.float32) + local_partial(c)
            comm[recv_slot] = acc.astype(jnp.bfloat16)

        hop(s, rs_process)

    own = lax.rem(me + 1, N_DEV)
    store_chunk(own, (N_DEV - 1) % 2)

    for t in range(N_DEV - 1):
        k = (N_DEV - 1) + t
        c_recv = lax.rem(me - t + 2 * N_DEV, N_DEV)

        def ag_process(recv_slot, c=c_recv):
            store_chunk(c, recv_slot)

        hop(k, ag_process)


def kernel(x, w_mat):
    out_shape = jax.ShapeDtypeStruct((M, N_OUT), jnp.float32)
    return pl.pallas_call(
        _body,
        out_shape=out_shape,
        in_specs=[
            pl.BlockSpec(memory_space=pltpu.MemorySpace.VMEM),
            pl.BlockSpec(memory_space=pltpu.MemorySpace.VMEM),
        ],
        out_specs=pl.BlockSpec(memory_space=pl.ANY),
        scratch_shapes=[
            pltpu.VMEM((2, CHUNK, N_OUT), jnp.bfloat16),
            pltpu.SemaphoreType.DMA((2,)),
            pltpu.SemaphoreType.DMA((2,)),
            pltpu.SemaphoreType.REGULAR,
            pltpu.SemaphoreType.DMA,
            pltpu.VMEM((CHUNK, N_OUT), jnp.float32),
        ],
        compiler_params=pltpu.CompilerParams(collective_id=0),
    )(x, w_mat)


# device time: 914654 ns/iter; 2.2060x vs baseline; 2.2060x over previous
import jax
import jax.numpy as jnp
from jax import lax
from jax.experimental import pallas as pl
from jax.experimental.pallas import tpu as pltpu

N_DEV = 32
M = 4096
N_OUT = 8192
CHUNK = M // N_DEV
HALF = CHUNK // 2
SLOTS = 4
HOPS = 2 * (N_DEV - 1)

_YZ_PATH = [(0, 0), (1, 0), (2, 0), (3, 0), (3, 1), (3, 2), (3, 3),
            (2, 3), (2, 2), (2, 1), (1, 1), (1, 2), (1, 3), (0, 3),
            (0, 2), (0, 1)]
_RING_COORDS = [(0, y, z) for (y, z) in _YZ_PATH] + \
               [(1, y, z) for (y, z) in reversed(_YZ_PATH)]
assert len(set(_RING_COORDS)) == N_DEV
for _i in range(N_DEV):
    _a, _b = _RING_COORDS[_i], _RING_COORDS[(_i + 1) % N_DEV]
    assert sum(abs(_a[_d] - _b[_d]) for _d in range(3)) == 1, (_a, _b)

_PLANE_Q = [(0, 0), (1, 0), (1, 1), (0, 1), (0, 2), (1, 2), (1, 3), (0, 3)]
_LOGICAL_COORDS = [(_PLANE_Q[p % 8][0], _PLANE_Q[p % 8][1], p // 8)
                   for p in range(N_DEV)]
_POS_OF_LOGICAL = [_RING_COORDS.index(c) for c in _LOGICAL_COORDS]
_RING_LOGICAL = [_LOGICAL_COORDS.index(c) for c in _RING_COORDS]


def _body(x_ref, w_ref, pos_tbl, ring_tbl, out_ref,
          comm_cw, comm_ccw, send_cw, recv_cw, send_ccw, recv_ccw,
          credit_cw, credit_ccw, store_sems, lp_buf, stage):
    me = lax.axis_index("i")
    r = pos_tbl[me]
    right = ring_tbl[lax.rem(r + 1, N_DEV)]
    left = ring_tbl[lax.rem(r + N_DEV - 1, N_DEV)]

    bsem = pltpu.get_barrier_semaphore()
    for nbr in (left, right):
        pl.semaphore_signal(bsem, inc=1, device_id=(nbr,),
                            device_id_type=pl.DeviceIdType.MESH)
    pl.semaphore_wait(bsem, 2)

    f32 = jnp.float32
    bf16 = jnp.bfloat16

    def lp_half(c, half):
        start = c * CHUNK + half * HALF
        return jnp.dot(x_ref[pl.ds(start, HALF), :], w_ref[...],
                       preferred_element_type=f32)

    store_state = {"count": 0, "pending": [None] * SLOTS}

    def store_half(c, half, vals_bf16):
        i = store_state["count"] % SLOTS
        if store_state["pending"][i] is not None:
            store_state["pending"][i].wait()
        stage[i] = jnp.maximum(vals_bf16.astype(f32), 0.0)
        cp = pltpu.make_async_copy(
            stage.at[i],
            out_ref.at[pl.ds(c * CHUNK + half * HALF, HALF), :],
            store_sems.at[i])
        cp.start()
        store_state["pending"][i] = cp
        store_state["count"] += 1

    def hop_rdmas(k):
        ss, rs = k % SLOTS, (k + 1) % SLOTS
        cw = pltpu.make_async_remote_copy(
            src_ref=comm_cw.at[ss], dst_ref=comm_cw.at[rs],
            send_sem=send_cw.at[ss], recv_sem=recv_cw.at[rs],
            device_id=(right,), device_id_type=pl.DeviceIdType.MESH)
        ccw = pltpu.make_async_remote_copy(
            src_ref=comm_ccw.at[ss], dst_ref=comm_ccw.at[rs],
            send_sem=send_ccw.at[ss], recv_sem=recv_ccw.at[rs],
            device_id=(left,), device_id_type=pl.DeviceIdType.MESH)
        if k >= SLOTS - 1:
            pl.semaphore_wait(credit_cw, 1)
            pl.semaphore_wait(credit_ccw, 1)
        cw.start()
        ccw.start()
        return cw, ccw

    def hop_credits(k):
        if k < HOPS - (SLOTS - 1):
            pl.semaphore_signal(credit_cw, inc=1, device_id=(left,),
                                device_id_type=pl.DeviceIdType.MESH)
            pl.semaphore_signal(credit_ccw, inc=1, device_id=(right,),
                                device_id_type=pl.DeviceIdType.MESH)

    lp0 = jnp.dot(x_ref[pl.ds(r * CHUNK, CHUNK), :], w_ref[...],
                  preferred_element_type=f32)
    comm_cw[0] = lp0[:HALF, :].astype(bf16)
    comm_ccw[0] = lp0[HALF:, :].astype(bf16)
    lp_buf[0, 0] = lp_half(lax.rem(r - 1 + N_DEV, N_DEV), 0)
    lp_buf[0, 1] = lp_half(lax.rem(r + 1, N_DEV), 1)

    for s in range(N_DEV - 1):
        rs = (s + 1) % SLOTS
        cw, ccw = hop_rdmas(s)
        if s + 1 < N_DEV - 1:
            lp_buf[(s + 1) % 2, 0] = lp_half(
                lax.rem(r - (s + 2) + 2 * N_DEV, N_DEV), 0)
            lp_buf[(s + 1) % 2, 1] = lp_half(
                lax.rem(r + (s + 2), N_DEV), 1)
        cw.wait()
        comm_cw[rs] = (comm_cw[rs].astype(f32)
                       + lp_buf[s % 2, 0]).astype(bf16)
        ccw.wait()
        comm_ccw[rs] = (comm_ccw[rs].astype(f32)
                        + lp_buf[s % 2, 1]).astype(bf16)
        hop_credits(s)

    own_cw = lax.rem(r + 1, N_DEV)
    own_ccw = lax.rem(r + N_DEV - 1, N_DEV)

    for t in range(N_DEV - 1):
        k = (N_DEV - 1) + t
        ss, rs = k % SLOTS, (k + 1) % SLOTS
        cw, ccw = hop_rdmas(k)
        if t == 0:
            store_half(own_cw, 0, comm_cw[ss])
            store_half(own_ccw, 1, comm_ccw[ss])
        else:
            store_half(lax.rem(r - (t - 1) + 2 * N_DEV, N_DEV), 0,
                       comm_cw[ss])
            store_half(lax.rem(r + (t - 1), N_DEV), 1, comm_ccw[ss])
        cw.wait()
        ccw.wait()
        hop_credits(k)

    last = (HOPS - 1 + 1) % SLOTS
    store_half(lax.rem(r - (N_DEV - 2) + 2 * N_DEV, N_DEV), 0,
               comm_cw[last])
    store_half(lax.rem(r + (N_DEV - 2), N_DEV), 1, comm_ccw[last])

    for cp in store_state["pending"]:
        if cp is not None:
            cp.wait()


def kernel(x, w_mat):
    pos_tbl = jnp.asarray(_POS_OF_LOGICAL, dtype=jnp.int32)
    ring_tbl = jnp.asarray(_RING_LOGICAL, dtype=jnp.int32)
    out_shape = jax.ShapeDtypeStruct((M, N_OUT), jnp.float32)
    return pl.pallas_call(
        _body,
        out_shape=out_shape,
        in_specs=[
            pl.BlockSpec(memory_space=pltpu.MemorySpace.VMEM),
            pl.BlockSpec(memory_space=pltpu.MemorySpace.VMEM),
            pl.BlockSpec(memory_space=pltpu.MemorySpace.SMEM),
            pl.BlockSpec(memory_space=pltpu.MemorySpace.SMEM),
        ],
        out_specs=pl.BlockSpec(memory_space=pl.ANY),
        scratch_shapes=[
            pltpu.VMEM((SLOTS, HALF, N_OUT), jnp.bfloat16),
            pltpu.VMEM((SLOTS, HALF, N_OUT), jnp.bfloat16),
            pltpu.SemaphoreType.DMA((SLOTS,)),
            pltpu.SemaphoreType.DMA((SLOTS,)),
            pltpu.SemaphoreType.DMA((SLOTS,)),
            pltpu.SemaphoreType.DMA((SLOTS,)),
            pltpu.SemaphoreType.REGULAR,
            pltpu.SemaphoreType.REGULAR,
            pltpu.SemaphoreType.DMA((SLOTS,)),
            pltpu.VMEM((2, 2, HALF, N_OUT), jnp.float32),
            pltpu.VMEM((SLOTS, HALF, N_OUT), jnp.float32),
        ],
        compiler_params=pltpu.CompilerParams(collective_id=0),
    )(x, w_mat, pos_tbl, ring_tbl)


# device time: 798235 ns/iter; 2.5277x vs baseline; 1.1458x over previous
import jax
import jax.numpy as jnp
from jax import lax
from jax.experimental import pallas as pl
from jax.experimental.pallas import tpu as pltpu

N_DEV = 32
M = 4096
N_OUT = 8192
CHUNK = M // N_DEV
HALF = CHUNK // 2
SUB = HALF // 2
SLOTS = 4
HOPS = 2 * (N_DEV - 1)
RS_HOPS = N_DEV - 1

_YZ_PATH = [(0, 0), (1, 0), (2, 0), (3, 0), (3, 1), (3, 2), (3, 3),
            (2, 3), (2, 2), (2, 1), (1, 1), (1, 2), (1, 3), (0, 3),
            (0, 2), (0, 1)]
_RING_COORDS = [(0, y, z) for (y, z) in _YZ_PATH] + \
               [(1, y, z) for (y, z) in reversed(_YZ_PATH)]
assert len(set(_RING_COORDS)) == N_DEV
for _i in range(N_DEV):
    _a, _b = _RING_COORDS[_i], _RING_COORDS[(_i + 1) % N_DEV]
    assert sum(abs(_a[_d] - _b[_d]) for _d in range(3)) == 1, (_a, _b)

_PLANE_Q = [(0, 0), (1, 0), (1, 1), (0, 1), (0, 2), (1, 2), (1, 3), (0, 3)]
_LOGICAL_COORDS = [(_PLANE_Q[p % 8][0], _PLANE_Q[p % 8][1], p // 8)
                   for p in range(N_DEV)]
_POS_OF_LOGICAL = [_RING_COORDS.index(c) for c in _LOGICAL_COORDS]
_RING_LOGICAL = [_LOGICAL_COORDS.index(c) for c in _RING_COORDS]


def _body(x_ref, w_ref, pos_tbl, ring_tbl, out_ref,
          comm_cw, comm_ccw, send_cw, recv_cw, send_ccw, recv_ccw,
          credit_cw, credit_ccw, store_sems, lp_buf, stage):
    me = lax.axis_index("i")
    r = pos_tbl[me]
    right = ring_tbl[lax.rem(r + 1, N_DEV)]
    left = ring_tbl[lax.rem(r + N_DEV - 1, N_DEV)]

    bsem = pltpu.get_barrier_semaphore()
    for nbr in (left, right):
        pl.semaphore_signal(bsem, inc=1, device_id=(nbr,),
                            device_id_type=pl.DeviceIdType.MESH)
    pl.semaphore_wait(bsem, 2)

    f32 = jnp.float32
    bf16 = jnp.bfloat16

    def lp_half(c, half):
        start = c * CHUNK + half * HALF
        return jnp.dot(x_ref[pl.ds(start, HALF), :], w_ref[...],
                       preferred_element_type=f32)

    def c_recv_cw(k):
        return lax.rem(r - (k + 1) + 2 * N_DEV, N_DEV)

    def c_recv_ccw(k):
        return lax.rem(r + (k + 1), N_DEV)

    store_state = {"count": 0, "pending": [None] * SLOTS}

    def store_half(c, half, vals_bf16):
        i = store_state["count"] % SLOTS
        if store_state["pending"][i] is not None:
            store_state["pending"][i].wait()
        stage[i] = jnp.maximum(vals_bf16.astype(f32), 0.0)
        cp = pltpu.make_async_copy(
            stage.at[i],
            out_ref.at[pl.ds(c * CHUNK + half * HALF, HALF), :],
            store_sems.at[i])
        cp.start()
        store_state["pending"][i] = cp
        store_state["count"] += 1

    def make_rdmas(k):
        ss, rs = k % SLOTS, (k + 1) % SLOTS
        out = {}
        for j in range(2):
            rows = pl.ds(j * SUB, SUB)
            out["cw", j] = pltpu.make_async_remote_copy(
                src_ref=comm_cw.at[ss, rows, :],
                dst_ref=comm_cw.at[rs, rows, :],
                send_sem=send_cw.at[ss, j], recv_sem=recv_cw.at[rs, j],
                device_id=(right,), device_id_type=pl.DeviceIdType.MESH)
            out["ccw", j] = pltpu.make_async_remote_copy(
                src_ref=comm_ccw.at[ss, rows, :],
                dst_ref=comm_ccw.at[rs, rows, :],
                send_sem=send_ccw.at[ss, j], recv_sem=recv_ccw.at[rs, j],
                device_id=(left,), device_id_type=pl.DeviceIdType.MESH)
        return out

    def process_prev(k_prev, j):
        rs = (k_prev + 1) % SLOTS
        rows = pl.ds(j * SUB, SUB)
        if k_prev < RS_HOPS:
            lrows = pl.ds(j * SUB, SUB)
            comm_cw[rs, rows, :] = (
                comm_cw[rs, rows, :].astype(f32)
                + lp_buf[k_prev % 2, 0, lrows, :]).astype(bf16)
            comm_ccw[rs, rows, :] = (
                comm_ccw[rs, rows, :].astype(f32)
                + lp_buf[k_prev % 2, 1, lrows, :]).astype(bf16)

    lp0 = jnp.dot(x_ref[pl.ds(r * CHUNK, CHUNK), :], w_ref[...],
                  preferred_element_type=f32)
    comm_cw[0] = lp0[:HALF, :].astype(bf16)
    comm_ccw[0] = lp0[HALF:, :].astype(bf16)
    lp_buf[0, 0] = lp_half(c_recv_cw(0), 0)
    lp_buf[0, 1] = lp_half(c_recv_ccw(0), 1)

    pending = None
    for k in range(HOPS):
        descs = make_rdmas(k)
        if k >= SLOTS - 1:
            pl.semaphore_wait(credit_cw, 1)
            pl.semaphore_wait(credit_ccw, 1)
        for j in range(2):
            if pending is not None:
                pending["cw", j].wait()
                pending["ccw", j].wait()
                process_prev(k - 1, j)
            descs["cw", j].start()
            descs["ccw", j].start()
        if 1 <= k < HOPS - (SLOTS - 1) + 1:
            pl.semaphore_signal(credit_cw, inc=1, device_id=(left,),
                                device_id_type=pl.DeviceIdType.MESH)
            pl.semaphore_signal(credit_ccw, inc=1, device_id=(right,),
                                device_id_type=pl.DeviceIdType.MESH)
        if k + 1 < RS_HOPS:
            lp_buf[(k + 1) % 2, 0] = lp_half(c_recv_cw(k + 1), 0)
            lp_buf[(k + 1) % 2, 1] = lp_half(c_recv_ccw(k + 1), 1)
        if k == RS_HOPS:
            store_half(lax.rem(r + 1, N_DEV), 0, comm_cw[k % SLOTS])
            store_half(lax.rem(r + N_DEV - 1, N_DEV), 1, comm_ccw[k % SLOTS])
        elif k > RS_HOPS:
            t_prev = k - 1 - RS_HOPS
            store_half(lax.rem(r - t_prev + 2 * N_DEV, N_DEV), 0,
                       comm_cw[k % SLOTS])
            store_half(lax.rem(r + t_prev, N_DEV), 1, comm_ccw[k % SLOTS])
        pending = descs

    for j in range(2):
        pending["cw", j].wait()
        pending["ccw", j].wait()
    t_last = HOPS - 1 - RS_HOPS
    store_half(lax.rem(r - t_last + 2 * N_DEV, N_DEV), 0,
               comm_cw[HOPS % SLOTS])
    store_half(lax.rem(r + t_last, N_DEV), 1, comm_ccw[HOPS % SLOTS])
    for cp in store_state["pending"]:
        if cp is not None:
            cp.wait()


def kernel(x, w_mat):
    pos_tbl = jnp.asarray(_POS_OF_LOGICAL, dtype=jnp.int32)
    ring_tbl = jnp.asarray(_RING_LOGICAL, dtype=jnp.int32)
    out_shape = jax.ShapeDtypeStruct((M, N_OUT), jnp.float32)
    return pl.pallas_call(
        _body,
        out_shape=out_shape,
        in_specs=[
            pl.BlockSpec(memory_space=pltpu.MemorySpace.VMEM),
            pl.BlockSpec(memory_space=pltpu.MemorySpace.VMEM),
            pl.BlockSpec(memory_space=pltpu.MemorySpace.SMEM),
            pl.BlockSpec(memory_space=pltpu.MemorySpace.SMEM),
        ],
        out_specs=pl.BlockSpec(memory_space=pl.ANY),
        scratch_shapes=[
            pltpu.VMEM((SLOTS, HALF, N_OUT), jnp.bfloat16),
            pltpu.VMEM((SLOTS, HALF, N_OUT), jnp.bfloat16),
            pltpu.SemaphoreType.DMA((SLOTS, 2)),
            pltpu.SemaphoreType.DMA((SLOTS, 2)),
            pltpu.SemaphoreType.DMA((SLOTS, 2)),
            pltpu.SemaphoreType.DMA((SLOTS, 2)),
            pltpu.SemaphoreType.REGULAR,
            pltpu.SemaphoreType.REGULAR,
            pltpu.SemaphoreType.DMA((SLOTS,)),
            pltpu.VMEM((2, 2, HALF, N_OUT), jnp.float32),
            pltpu.VMEM((SLOTS, HALF, N_OUT), jnp.float32),
        ],
        compiler_params=pltpu.CompilerParams(collective_id=0),
    )(x, w_mat, pos_tbl, ring_tbl)
